# baseline (speedup 1.0000x reference)
import functools

import jax
import jax.numpy as jnp
from jax import lax
from jax.experimental import pallas as pl
from jax.experimental.pallas import tpu as pltpu
from jax.experimental.pallas import tpu_sc as plsc

_D = 1024
_SEQ = 2048
_NC = 2
_NS = 16
_NL = 16
_NW = _NC * _NS
_CH = 32


def _sc_embed(ids_flat, word_table, pos_table):
    tok = ids_flat.shape[0]
    bpw = tok // _NW
    nchunk = bpw // _CH
    mesh = plsc.VectorSubcoreMesh(core_axis_name="c", subcore_axis_name="s")

    @functools.partial(
        pl.kernel,
        mesh=mesh,
        out_type=jax.ShapeDtypeStruct((tok, _D), jnp.float32),
        scratch_types=[
            pltpu.VMEM((_CH,), jnp.int32),
            pltpu.VMEM((_CH, _D), jnp.float32),
            pltpu.VMEM((_CH, _D), jnp.float32),
            pltpu.SemaphoreType.DMA,
        ],
    )
    def k(ids_hbm, wt_hbm, pt_hbm, out_hbm, idx_v, rows_v, pos_v, sem):
        wid = lax.axis_index("s") * _NC + lax.axis_index("c")
        base = wid * bpw

        @pl.loop(0, nchunk)
        def _(c):
            off = base + c * _CH
            pos_off = lax.rem(off, _SEQ)
            pltpu.sync_copy(ids_hbm.at[pl.ds(off, _CH)], idx_v)
            pltpu.async_copy(wt_hbm.at[idx_v], rows_v, sem).wait()
            pltpu.sync_copy(pt_hbm.at[pl.ds(pos_off, _CH)], pos_v)

            @pl.loop(0, _CH)
            def _(r):
                @pl.loop(0, _D, step=_NL)
                def _(c0):
                    slc = (r, pl.ds(c0, _NL))
                    rows_v[slc] = rows_v[slc] + pos_v[slc]

            pltpu.sync_copy(rows_v, out_hbm.at[pl.ds(off, _CH)])

    return k(ids_flat, word_table, pos_table)


def kernel(input_ids, word_table, pos_table):
    b, s = input_ids.shape
    ids_flat = input_ids.reshape(-1).astype(jnp.int32)
    out = _sc_embed(ids_flat, word_table, pos_table)
    return out.reshape(b, s, _D)


# speedup vs baseline: 2.3816x; 2.3816x over previous
import functools

import jax
import jax.numpy as jnp
from jax import lax
from jax.experimental import pallas as pl
from jax.experimental.pallas import tpu as pltpu
from jax.experimental.pallas import tpu_sc as plsc

_D = 1024
_SEQ = 2048
_NC = 2
_NS = 16
_NL = 16
_NW = _NC * _NS
_CH = 16
_UNROLL = 4


def _sc_embed(ids_flat, word_table, pos_table):
    tok = ids_flat.shape[0]
    bpw = tok // _NW
    nchunk = bpw // _CH
    mesh = plsc.VectorSubcoreMesh(core_axis_name="c", subcore_axis_name="s")

    @functools.partial(
        pl.kernel,
        mesh=mesh,
        out_type=jax.ShapeDtypeStruct((tok, _D), jnp.float32),
        scratch_types=[
            pltpu.VMEM((bpw,), jnp.int32),
            pltpu.VMEM((2, _CH, _D), jnp.float32),
            pltpu.VMEM((2, _CH, _D), jnp.float32),
            pltpu.VMEM((2, _CH, _D), jnp.float32),
            pltpu.SemaphoreType.DMA,
            pltpu.SemaphoreType.DMA,
            pltpu.SemaphoreType.DMA,
            pltpu.SemaphoreType.DMA,
            pltpu.SemaphoreType.DMA,
            pltpu.SemaphoreType.DMA,
        ],
    )
    def k(ids_hbm, wt_hbm, pt_hbm, out_hbm, idx_v, rows2, pos2, out2,
          g0, g1, p0, p1, w0, w1):
        gs = (g0, g1)
        ps = (p0, p1)
        ws = (w0, w1)
        wid = lax.axis_index("s") * _NC + lax.axis_index("c")
        base = wid * bpw
        pos_base = lax.rem(base, _SEQ)

        pltpu.sync_copy(ids_hbm.at[pl.ds(base, bpw)], idx_v)

        def start(c, b):
            pltpu.async_copy(
                wt_hbm.at[idx_v.at[pl.ds(c * _CH, _CH)]], rows2.at[b], gs[b])
            pltpu.async_copy(
                pt_hbm.at[pl.ds(pos_base + c * _CH, _CH)], pos2.at[b], ps[b])

        for b in range(2):
            start(b, b)

        @pl.loop(0, nchunk, step=2)
        def _(c):
            for b in range(2):
                cc = c + b
                pltpu.make_async_copy(
                    wt_hbm.at[pl.ds(0, _CH)], rows2.at[b], gs[b]).wait()
                pltpu.make_async_copy(
                    pt_hbm.at[pl.ds(0, _CH)], pos2.at[b], ps[b]).wait()

                @pl.when(cc >= 2)
                def _():
                    pltpu.make_async_copy(
                        wt_hbm.at[pl.ds(0, _CH)], out2.at[b], ws[b]).wait()

                rb = rows2.at[b]
                pb = pos2.at[b]
                ob = out2.at[b]

                @pl.loop(0, _CH)
                def _(r):
                    @pl.loop(0, _D, step=_UNROLL * _NL)
                    def _(c0):
                        for u in range(_UNROLL):
                            slc = (r, pl.ds(c0 + u * _NL, _NL))
                            ob[slc] = rb[slc] + pb[slc]

                pltpu.async_copy(
                    ob, out_hbm.at[pl.ds(base + cc * _CH, _CH)], ws[b])

                @pl.when(cc + 2 < nchunk)
                def _():
                    start(cc + 2, b)

        for b in range(2):
            pltpu.make_async_copy(
                wt_hbm.at[pl.ds(0, _CH)], out2.at[b], ws[b]).wait()

    return k(ids_flat, word_table, pos_table)


def kernel(input_ids, word_table, pos_table):
    b, s = input_ids.shape
    ids_flat = input_ids.reshape(-1).astype(jnp.int32)
    out = _sc_embed(ids_flat, word_table, pos_table)
    return out.reshape(b, s, _D)
